# baseline (speedup 1.0000x reference)
import jax
import jax.numpy as jnp
from jax import lax
from jax.experimental import pallas as pl
from jax.experimental.pallas import tpu as pltpu

_G = 2
_V = 320
_D = 256
_TILE = 4096
_SUB = 512


def _vq_body(x_ref, w_ref, b_ref, cv_ref, out_ref, perp_ref, acc_ref):
    i = pl.program_id(0)
    n = pl.num_programs(0)

    iota = lax.broadcasted_iota(
        jnp.int32, (_SUB, _V), 1).astype(jnp.float32)
    w = w_ref[...]
    bias = b_ref[...]
    counts = [jnp.zeros((1, _V), jnp.float32) for _ in range(_G)]
    for k in range(_TILE // _SUB):
        x = x_ref[pl.ds(k * _SUB, _SUB), :]
        logits = lax.dot_general(
            x, w, (((1,), (1,)), ((), ()))
        ) + bias
        outs = []
        for g in range(_G):
            lg = logits[:, g * _V:(g + 1) * _V]
            m = jnp.max(lg, axis=1, keepdims=True)
            cand = jnp.where(lg == m, iota, jnp.float32(_V))
            idx = jnp.min(cand, axis=1, keepdims=True)
            oh = (iota == idx).astype(jnp.float32)
            cvg = cv_ref[g * _V:(g + 1) * _V, :]
            outs.append(jnp.dot(oh, cvg))
            counts[g] = counts[g] + jnp.sum(oh, axis=0, keepdims=True)
        out_ref[pl.ds(k * _SUB, _SUB), :] = jnp.concatenate(outs, axis=1)

    @pl.when(i == 0)
    def _():
        acc_ref[...] = jnp.zeros_like(acc_ref)

    acc_ref[...] += jnp.concatenate(counts, axis=0)

    @pl.when(i == n - 1)
    def _():
        p = acc_ref[...] / jnp.float32(n * _TILE)
        ent = jnp.sum(p * jnp.log(p + 1e-7), axis=1, keepdims=True)
        perp_ref[...] = jnp.sum(jnp.exp(-ent), keepdims=True)


def kernel(hidden_states, W, b, codevectors):
    B, S, H = hidden_states.shape
    N = B * S
    x = hidden_states.reshape(N, H)
    cv = codevectors.reshape(_G * _V, _D)
    b2 = b.reshape(1, _G * _V)

    out, perp = pl.pallas_call(
        _vq_body,
        grid=(N // _TILE,),
        in_specs=[
            pl.BlockSpec((_TILE, H), lambda i: (i, 0)),
            pl.BlockSpec((_G * _V, H), lambda i: (0, 0)),
            pl.BlockSpec((1, _G * _V), lambda i: (0, 0)),
            pl.BlockSpec((_G * _V, _D), lambda i: (0, 0)),
        ],
        out_specs=[
            pl.BlockSpec((_TILE, _G * _D), lambda i: (i, 0)),
            pl.BlockSpec((1, 1), lambda i: (0, 0)),
        ],
        out_shape=[
            jax.ShapeDtypeStruct((N, _G * _D), jnp.float32),
            jax.ShapeDtypeStruct((1, 1), jnp.float32),
        ],
        scratch_shapes=[pltpu.VMEM((_G, _V), jnp.float32)],
        compiler_params=pltpu.CompilerParams(
            dimension_semantics=("arbitrary",)
        ),
    )(x, W, b2, cv)
    return out.reshape(B, S, _G * _D), perp[0, 0]


# speedup vs baseline: 1.0814x; 1.0814x over previous
import jax
import jax.numpy as jnp
from jax import lax
from jax.experimental import pallas as pl
from jax.experimental.pallas import tpu as pltpu

_G = 2
_V = 320
_D = 256
_TILE = 2048
_SUB = 512


def _vq_body(x_ref, w_ref, b_ref, cv_ref, out_ref, perp_ref, acc_ref):
    i = pl.program_id(0)
    n = pl.num_programs(0)

    iota = lax.broadcasted_iota(
        jnp.int32, (_SUB, _V), 1).astype(jnp.float32)
    w = w_ref[...]
    bias = b_ref[...]
    counts = [jnp.zeros((1, _V), jnp.float32) for _ in range(_G)]
    for k in range(_TILE // _SUB):
        x = x_ref[pl.ds(k * _SUB, _SUB), :]
        logits = lax.dot_general(
            x, w, (((1,), (1,)), ((), ()))
        ) + bias
        outs = []
        for g in range(_G):
            lg = logits[:, g * _V:(g + 1) * _V]
            m = jnp.max(lg, axis=1, keepdims=True)
            cand = jnp.where(lg == m, iota, jnp.float32(_V))
            idx = jnp.min(cand, axis=1, keepdims=True)
            oh = (iota == idx).astype(jnp.float32)
            cvg = cv_ref[g * _V:(g + 1) * _V, :]
            outs.append(jnp.dot(oh, cvg))
            counts[g] = counts[g] + jnp.sum(oh, axis=0, keepdims=True)
        out_ref[pl.ds(k * _SUB, _SUB), :] = jnp.concatenate(outs, axis=1)

    @pl.when(i == 0)
    def _():
        acc_ref[...] = jnp.zeros_like(acc_ref)

    acc_ref[...] += jnp.concatenate(counts, axis=0)

    @pl.when(i == n - 1)
    def _():
        p = acc_ref[...] / jnp.float32(n * _TILE)
        ent = jnp.sum(p * jnp.log(p + 1e-7), axis=1, keepdims=True)
        perp_ref[...] = jnp.sum(jnp.exp(-ent), keepdims=True)


def kernel(hidden_states, W, b, codevectors):
    B, S, H = hidden_states.shape
    N = B * S
    x = hidden_states.reshape(N, H)
    cv = codevectors.reshape(_G * _V, _D)
    b2 = b.reshape(1, _G * _V)

    out, perp = pl.pallas_call(
        _vq_body,
        grid=(N // _TILE,),
        in_specs=[
            pl.BlockSpec((_TILE, H), lambda i: (i, 0)),
            pl.BlockSpec((_G * _V, H), lambda i: (0, 0)),
            pl.BlockSpec((1, _G * _V), lambda i: (0, 0)),
            pl.BlockSpec((_G * _V, _D), lambda i: (0, 0)),
        ],
        out_specs=[
            pl.BlockSpec((_TILE, _G * _D), lambda i: (i, 0)),
            pl.BlockSpec((1, 1), lambda i: (0, 0)),
        ],
        out_shape=[
            jax.ShapeDtypeStruct((N, _G * _D), jnp.float32),
            jax.ShapeDtypeStruct((1, 1), jnp.float32),
        ],
        scratch_shapes=[pltpu.VMEM((_G, _V), jnp.float32)],
        compiler_params=pltpu.CompilerParams(
            dimension_semantics=("arbitrary",)
        ),
    )(x, W, b2, cv)
    return out.reshape(B, S, _G * _D), perp[0, 0]


# speedup vs baseline: 1.1213x; 1.0369x over previous
import jax
import jax.numpy as jnp
from jax import lax
from jax.experimental import pallas as pl
from jax.experimental.pallas import tpu as pltpu

_G = 2
_V = 320
_D = 256
_TILE = 2048
_SUB = 1024


def _vq_body(x_ref, w_ref, b_ref, cv_ref, out_ref, perp_ref, acc_ref):
    i = pl.program_id(0)
    n = pl.num_programs(0)

    iota = lax.broadcasted_iota(
        jnp.int32, (_SUB, _V), 1).astype(jnp.float32)
    w = w_ref[...]
    bias = b_ref[...]
    counts = [jnp.zeros((1, _V), jnp.float32) for _ in range(_G)]
    for k in range(_TILE // _SUB):
        x = x_ref[pl.ds(k * _SUB, _SUB), :]
        logits = lax.dot_general(
            x, w, (((1,), (1,)), ((), ()))
        ) + bias
        outs = []
        for g in range(_G):
            lg = logits[:, g * _V:(g + 1) * _V]
            m = jnp.max(lg, axis=1, keepdims=True)
            cand = jnp.where(lg == m, iota, jnp.float32(_V))
            idx = jnp.min(cand, axis=1, keepdims=True)
            oh = (iota == idx).astype(jnp.float32)
            cvg = cv_ref[g * _V:(g + 1) * _V, :]
            outs.append(jnp.dot(oh, cvg))
            counts[g] = counts[g] + jnp.sum(oh, axis=0, keepdims=True)
        out_ref[pl.ds(k * _SUB, _SUB), :] = jnp.concatenate(outs, axis=1)

    @pl.when(i == 0)
    def _():
        acc_ref[...] = jnp.zeros_like(acc_ref)

    acc_ref[...] += jnp.concatenate(counts, axis=0)

    @pl.when(i == n - 1)
    def _():
        p = acc_ref[...] / jnp.float32(n * _TILE)
        ent = jnp.sum(p * jnp.log(p + 1e-7), axis=1, keepdims=True)
        perp_ref[...] = jnp.sum(jnp.exp(-ent), keepdims=True)


def kernel(hidden_states, W, b, codevectors):
    B, S, H = hidden_states.shape
    N = B * S
    x = hidden_states.reshape(N, H)
    cv = codevectors.reshape(_G * _V, _D)
    b2 = b.reshape(1, _G * _V)

    out, perp = pl.pallas_call(
        _vq_body,
        grid=(N // _TILE,),
        in_specs=[
            pl.BlockSpec((_TILE, H), lambda i: (i, 0)),
            pl.BlockSpec((_G * _V, H), lambda i: (0, 0)),
            pl.BlockSpec((1, _G * _V), lambda i: (0, 0)),
            pl.BlockSpec((_G * _V, _D), lambda i: (0, 0)),
        ],
        out_specs=[
            pl.BlockSpec((_TILE, _G * _D), lambda i: (i, 0)),
            pl.BlockSpec((1, 1), lambda i: (0, 0)),
        ],
        out_shape=[
            jax.ShapeDtypeStruct((N, _G * _D), jnp.float32),
            jax.ShapeDtypeStruct((1, 1), jnp.float32),
        ],
        scratch_shapes=[pltpu.VMEM((_G, _V), jnp.float32)],
        compiler_params=pltpu.CompilerParams(
            dimension_semantics=("arbitrary",)
        ),
    )(x, W, b2, cv)
    return out.reshape(B, S, _G * _D), perp[0, 0]


# speedup vs baseline: 1.1219x; 1.0005x over previous
import jax
import jax.numpy as jnp
from jax import lax
from jax.experimental import pallas as pl
from jax.experimental.pallas import tpu as pltpu

_G = 2
_V = 320
_D = 256
_TILE = 2048
_SUB = 1024


def _vq_body(x_ref, w_ref, b_ref, cv_ref, out_ref, perp_ref, acc_ref):
    i = pl.program_id(0)
    n = pl.num_programs(0)

    iota = lax.broadcasted_iota(
        jnp.int32, (_SUB, _V), 1).astype(jnp.float32)
    w = w_ref[...]
    bias = b_ref[...]
    counts = [jnp.zeros((1, _V), jnp.float32) for _ in range(_G)]
    for k in range(_TILE // _SUB):
        x = x_ref[pl.ds(k * _SUB, _SUB), :]
        logits = lax.dot_general(
            x, w, (((1,), (1,)), ((), ()))
        ) + bias
        outs = []
        for g in range(_G):
            lg = logits[:, g * _V:(g + 1) * _V]
            m = jnp.max(lg, axis=1, keepdims=True)
            cand = jnp.where(lg == m, iota, jnp.float32(_V))
            idx = jnp.min(cand, axis=1, keepdims=True)
            oh = (iota == idx).astype(jnp.float32)
            cvg = cv_ref[g * _V:(g + 1) * _V, :]
            outs.append(jnp.dot(oh, cvg))
            counts[g] = counts[g] + jnp.sum(oh, axis=0, keepdims=True)
        for g in range(_G):
            out_ref[pl.ds(k * _SUB, _SUB), pl.ds(g * _D, _D)] = outs[g]

    @pl.when(i == 0)
    def _():
        acc_ref[...] = jnp.zeros_like(acc_ref)

    acc_ref[...] += jnp.concatenate(counts, axis=0)

    @pl.when(i == n - 1)
    def _():
        p = acc_ref[...] / jnp.float32(n * _TILE)
        ent = jnp.sum(p * jnp.log(p + 1e-7), axis=1, keepdims=True)
        perp_ref[...] = jnp.sum(jnp.exp(-ent), keepdims=True)


def kernel(hidden_states, W, b, codevectors):
    B, S, H = hidden_states.shape
    N = B * S
    x = hidden_states.reshape(N, H)
    cv = codevectors.reshape(_G * _V, _D)
    b2 = b.reshape(1, _G * _V)

    out, perp = pl.pallas_call(
        _vq_body,
        grid=(N // _TILE,),
        in_specs=[
            pl.BlockSpec((_TILE, H), lambda i: (i, 0)),
            pl.BlockSpec((_G * _V, H), lambda i: (0, 0)),
            pl.BlockSpec((1, _G * _V), lambda i: (0, 0)),
            pl.BlockSpec((_G * _V, _D), lambda i: (0, 0)),
        ],
        out_specs=[
            pl.BlockSpec((_TILE, _G * _D), lambda i: (i, 0)),
            pl.BlockSpec((1, 1), lambda i: (0, 0)),
        ],
        out_shape=[
            jax.ShapeDtypeStruct((N, _G * _D), jnp.float32),
            jax.ShapeDtypeStruct((1, 1), jnp.float32),
        ],
        scratch_shapes=[pltpu.VMEM((_G, _V), jnp.float32)],
        compiler_params=pltpu.CompilerParams(
            dimension_semantics=("arbitrary",)
        ),
    )(x, W, b2, cv)
    return out.reshape(B, S, _G * _D), perp[0, 0]
